# baseline (speedup 1.0000x reference)
import math

import jax
import jax.numpy as jnp
from jax.experimental import pallas as pl
from jax.experimental.pallas import tpu as pltpu

B = 16
Z = 4096
D = 768
O = 32
ZC = 1024
NZ = Z // ZC
_SCALE = 1.0 / math.sqrt(D)


def _zoner_kernel(txt_ref, zone_ref, wt_ref, bt_ref, wz_ref, bz_ref,
                  mask_ref, out_ref):
    b = pl.program_id(0)
    zb = pl.program_id(1)
    t = jnp.tanh(
        jnp.dot(txt_ref[pl.ds(b, 1), :], wt_ref[...],
                preferred_element_type=jnp.float32) + bt_ref[...])
    z = jnp.tanh(
        jnp.dot(zone_ref[0], wz_ref[...],
                preferred_element_type=jnp.float32) + bz_ref[...])
    logits = jax.lax.dot_general(
        t, z, (((1,), (1,)), ((), ())),
        preferred_element_type=jnp.float32) * _SCALE
    logits = jnp.where(mask_ref[0, 0], -jnp.inf, logits)
    out_ref[0, :, pl.ds(zb * ZC, ZC)] = logits

    @pl.when(zb == NZ - 1)
    def _softmax():
        row = out_ref[0]
        m = jnp.max(row, axis=1, keepdims=True)
        e = jnp.exp(row - m)
        out_ref[0] = e / jnp.sum(e, axis=1, keepdims=True)


def kernel(txt_embeds, zone_embeds, W_txt, b_txt, W_zone, b_zone, mask):
    wt = W_txt.T
    wz = W_zone.T
    bt = b_txt.reshape(1, O)
    bz = b_zone.reshape(1, O)
    mask3 = mask.reshape(B, NZ, 1, ZC)

    out = pl.pallas_call(
        _zoner_kernel,
        grid=(B, NZ),
        in_specs=[
            pl.BlockSpec((B, D), lambda b, zb: (0, 0)),
            pl.BlockSpec((1, ZC, D), lambda b, zb: (b, zb, 0)),
            pl.BlockSpec((D, O), lambda b, zb: (0, 0)),
            pl.BlockSpec((1, O), lambda b, zb: (0, 0)),
            pl.BlockSpec((D, O), lambda b, zb: (0, 0)),
            pl.BlockSpec((1, O), lambda b, zb: (0, 0)),
            pl.BlockSpec((1, 1, 1, ZC), lambda b, zb: (b, zb, 0, 0)),
        ],
        out_specs=pl.BlockSpec((1, 1, Z), lambda b, zb: (b, 0, 0)),
        out_shape=jax.ShapeDtypeStruct((B, 1, Z), jnp.float32),
        compiler_params=pltpu.CompilerParams(
            dimension_semantics=("arbitrary", "arbitrary")),
    )(txt_embeds, zone_embeds.reshape(B, NZ * ZC, D), wt, bt, wz, bz, mask3)
    return out.reshape(B, Z)


# speedup vs baseline: 1.2253x; 1.2253x over previous
import math

import jax
import jax.numpy as jnp
from jax.experimental import pallas as pl
from jax.experimental.pallas import tpu as pltpu

B = 16
Z = 4096
D = 768
O = 32
ZH = Z // 2
_SCALE = 1.0 / math.sqrt(D)


def _zoner_kernel(txt_ref, zone_a_ref, zone_b_ref, wt_ref, bt_ref, wz_ref,
                  bz_ref, mask_ref, out_ref):
    b = pl.program_id(0)
    t = jnp.tanh(
        jnp.dot(txt_ref[pl.ds(b, 1), :], wt_ref[...],
                preferred_element_type=jnp.float32) + bt_ref[...])

    def half_logits(zref):
        z = jnp.tanh(
            jnp.dot(zref[0, 0], wz_ref[...],
                    preferred_element_type=jnp.float32) + bz_ref[...])
        return jax.lax.dot_general(
            t, z, (((1,), (1,)), ((), ())),
            preferred_element_type=jnp.float32)

    logits = jnp.concatenate(
        [half_logits(zone_a_ref), half_logits(zone_b_ref)], axis=1) * _SCALE
    logits = jnp.where(mask_ref[0], -jnp.inf, logits)
    m = jnp.max(logits, axis=1, keepdims=True)
    e = jnp.exp(logits - m)
    out_ref[0] = e / jnp.sum(e, axis=1, keepdims=True)


def kernel(txt_embeds, zone_embeds, W_txt, b_txt, W_zone, b_zone, mask):
    wt = W_txt.T
    wz = W_zone.T
    bt = b_txt.reshape(1, O)
    bz = b_zone.reshape(1, O)
    mask3 = mask.reshape(B, 1, Z)
    zone4 = zone_embeds.reshape(B, 2, ZH, D)

    out = pl.pallas_call(
        _zoner_kernel,
        grid=(B,),
        in_specs=[
            pl.BlockSpec((B, D), lambda b: (0, 0)),
            pl.BlockSpec((1, 1, ZH, D), lambda b: (b, 0, 0, 0)),
            pl.BlockSpec((1, 1, ZH, D), lambda b: (b, 1, 0, 0)),
            pl.BlockSpec((D, O), lambda b: (0, 0)),
            pl.BlockSpec((1, O), lambda b: (0, 0)),
            pl.BlockSpec((D, O), lambda b: (0, 0)),
            pl.BlockSpec((1, O), lambda b: (0, 0)),
            pl.BlockSpec((1, 1, Z), lambda b: (b, 0, 0)),
        ],
        out_specs=pl.BlockSpec((1, 1, Z), lambda b: (b, 0, 0)),
        out_shape=jax.ShapeDtypeStruct((B, 1, Z), jnp.float32),
    )(txt_embeds, zone4, zone4, wt, bt, wz, bz, mask3)
    return out.reshape(B, Z)


# speedup vs baseline: 1.4216x; 1.1602x over previous
import math

import jax
import jax.numpy as jnp
from jax.experimental import pallas as pl
from jax.experimental.pallas import tpu as pltpu

B = 16
Z = 4096
D = 768
O = 32
_SCALE = 1.0 / math.sqrt(D)


def _zoner_kernel(txt_ref, zone_ref, wt_ref, bt_ref, wz_ref, bz_ref,
                  mask_ref, out_ref):
    b = pl.program_id(0)
    t = jnp.tanh(
        jnp.dot(txt_ref[pl.ds(b, 1), :], wt_ref[...],
                preferred_element_type=jnp.float32) + bt_ref[...])
    z = jnp.tanh(
        jnp.dot(zone_ref[0], wz_ref[...],
                preferred_element_type=jnp.float32) + bz_ref[...])
    logits = jax.lax.dot_general(
        t, z, (((1,), (1,)), ((), ())),
        preferred_element_type=jnp.float32) * _SCALE
    logits = jnp.where(mask_ref[pl.ds(b, 1), 0, :], -jnp.inf, logits)
    m = jnp.max(logits, axis=1, keepdims=True)
    e = jnp.exp(logits - m)
    out_ref[0] = e / jnp.sum(e, axis=1, keepdims=True)


def kernel(txt_embeds, zone_embeds, W_txt, b_txt, W_zone, b_zone, mask):
    wt = W_txt.T
    wz = W_zone.T
    bt = b_txt.reshape(1, O)
    bz = b_zone.reshape(1, O)
    mask3 = mask.reshape(B, 1, Z)

    out = pl.pallas_call(
        _zoner_kernel,
        grid=(B,),
        in_specs=[
            pl.BlockSpec((B, D), lambda b: (0, 0)),
            pl.BlockSpec((1, Z, D), lambda b: (b, 0, 0)),
            pl.BlockSpec((D, O), lambda b: (0, 0)),
            pl.BlockSpec((1, O), lambda b: (0, 0)),
            pl.BlockSpec((D, O), lambda b: (0, 0)),
            pl.BlockSpec((1, O), lambda b: (0, 0)),
            pl.BlockSpec((B, 1, Z), lambda b: (0, 0, 0)),
        ],
        out_specs=pl.BlockSpec((1, 1, Z), lambda b: (b, 0, 0)),
        out_shape=jax.ShapeDtypeStruct((B, 1, Z), jnp.float32),
    )(txt_embeds, zone_embeds, wt, bt, wz, bz, mask3)
    return out.reshape(B, Z)
